# baseline (speedup 1.0000x reference)
import functools

import jax
import jax.numpy as jnp
from jax.experimental import pallas as pl
from jax.experimental.pallas import tpu as pltpu

ALPHA = 0.1
N = 16384
D = 64
BM = 512
BK = 2048


def _prop_kernel(adj_ref, inp_ref, h_ref, out_ref):
    k = pl.program_id(1)

    @pl.when(k == 0)
    def _init():
        out_ref[...] = jnp.zeros_like(out_ref)

    out_ref[...] += jnp.dot(
        adj_ref[...], inp_ref[...], preferred_element_type=jnp.float32
    )

    @pl.when(k == pl.num_programs(1) - 1)
    def _finalize():
        out_ref[...] = (1.0 - ALPHA) * out_ref[...] + ALPHA * h_ref[...]


@functools.partial(jax.jit, static_argnames=())
def kernel(input, adj, h, W):
    del W
    grid = (N // BM, N // BK)
    return pl.pallas_call(
        _prop_kernel,
        grid=grid,
        in_specs=[
            pl.BlockSpec((BM, BK), lambda i, k: (i, k)),
            pl.BlockSpec((BK, D), lambda i, k: (k, 0)),
            pl.BlockSpec((BM, D), lambda i, k: (i, 0)),
        ],
        out_specs=pl.BlockSpec((BM, D), lambda i, k: (i, 0)),
        out_shape=jax.ShapeDtypeStruct((N, D), jnp.float32),
        compiler_params=pltpu.CompilerParams(
            dimension_semantics=("parallel", "arbitrary"),
        ),
    )(adj, input, h)


# speedup vs baseline: 1.3213x; 1.3213x over previous
import functools

import jax
import jax.numpy as jnp
from jax.experimental import pallas as pl
from jax.experimental.pallas import tpu as pltpu

ALPHA = 0.1
N = 16384
D = 64
BM = 256


def _prop_kernel(adj_ref, inp_ref, h_ref, out_ref):
    out_ref[...] = (1.0 - ALPHA) * jnp.dot(
        adj_ref[...], inp_ref[...], preferred_element_type=jnp.float32
    ) + ALPHA * h_ref[...]


@functools.partial(jax.jit, static_argnames=())
def kernel(input, adj, h, W):
    del W
    grid = (N // BM,)
    return pl.pallas_call(
        _prop_kernel,
        grid=grid,
        in_specs=[
            pl.BlockSpec((BM, N), lambda i: (i, 0)),
            pl.BlockSpec((N, D), lambda i: (0, 0)),
            pl.BlockSpec((BM, D), lambda i: (i, 0)),
        ],
        out_specs=pl.BlockSpec((BM, D), lambda i: (i, 0)),
        out_shape=jax.ShapeDtypeStruct((N, D), jnp.float32),
        compiler_params=pltpu.CompilerParams(
            dimension_semantics=("arbitrary",),
        ),
    )(adj, input, h)


# speedup vs baseline: 1.3373x; 1.0121x over previous
import functools

import jax
import jax.numpy as jnp
from jax.experimental import pallas as pl
from jax.experimental.pallas import tpu as pltpu

ALPHA = 0.1
N = 16384
D = 64
BM = 128


def _prop_kernel(adj_ref, inp_ref, h_ref, out_ref):
    out_ref[...] = (1.0 - ALPHA) * jnp.dot(
        adj_ref[...], inp_ref[...], preferred_element_type=jnp.float32
    ) + ALPHA * h_ref[...]


@functools.partial(jax.jit, static_argnames=())
def kernel(input, adj, h, W):
    del W
    grid = (N // BM,)
    return pl.pallas_call(
        _prop_kernel,
        grid=grid,
        in_specs=[
            pl.BlockSpec((BM, N), lambda i: (i, 0)),
            pl.BlockSpec((N, D), lambda i: (0, 0)),
            pl.BlockSpec((BM, D), lambda i: (i, 0)),
        ],
        out_specs=pl.BlockSpec((BM, D), lambda i: (i, 0)),
        out_shape=jax.ShapeDtypeStruct((N, D), jnp.float32),
        compiler_params=pltpu.CompilerParams(
            dimension_semantics=("arbitrary",),
        ),
    )(adj, input, h)
